# baseline (speedup 1.0000x reference)
import functools

import jax
import jax.numpy as jnp
from jax import lax
from jax.experimental import pallas as pl
from jax.experimental.pallas import tpu as pltpu
from jax.experimental.pallas import tpu_sc as plsc

NUM_CLASS = 100000
N_CLS_CTX = 4
CTX_DIM = 512
TOK_LEN = 77
BATCH = 4096
PREFIX_LEN = 5
SUFFIX_LEN = 68

ROW = N_CLS_CTX * CTX_DIM
PRE_W = PREFIX_LEN * CTX_DIM
SUF_W = SUFFIX_LEN * CTX_DIM
OUT_W = TOK_LEN * CTX_DIM

NUM_CORES = 2
NUM_SUBCORES = 16
NW = NUM_CORES * NUM_SUBCORES
BPW = BATCH // NW
CH = 16
NCH = BPW // CH


def _body(lab_hbm, table_hbm, pfx_hbm, sfx_hbm, out_hbm,
          idx_v, pfx_v, sfx_v, rows_v, sem):
    wid = lax.axis_index("s") * NUM_CORES + lax.axis_index("c")
    base = wid * BPW
    pltpu.sync_copy(lab_hbm.at[pl.ds(base, BPW)], idx_v)
    pltpu.sync_copy(pfx_hbm, pfx_v)
    pltpu.sync_copy(sfx_hbm, sfx_v)

    def chunk(c, _):
        cb = base + c * CH
        pltpu.async_copy(
            table_hbm.at[idx_v.at[pl.ds(c * CH, CH)]], rows_v, sem
        ).wait()
        pltpu.sync_copy(rows_v, out_hbm.at[pl.ds(cb, CH), pl.ds(PRE_W, ROW)])

        def elem(i, _):
            pltpu.sync_copy(pfx_v, out_hbm.at[pl.ds(cb + i, 1), pl.ds(0, PRE_W)])
            pltpu.sync_copy(
                sfx_v, out_hbm.at[pl.ds(cb + i, 1), pl.ds(PRE_W + ROW, SUF_W)]
            )
            return 0

        lax.fori_loop(0, CH, elem, 0)
        return 0

    lax.fori_loop(0, NCH, chunk, 0)


def kernel(label, cls_ctx, token_prefix, token_suffix):
    table = cls_ctx.reshape(NUM_CLASS, ROW)
    pfx = token_prefix.reshape(1, PRE_W)
    sfx = token_suffix.reshape(1, SUF_W)
    lab = label.astype(jnp.int32)

    mesh = plsc.VectorSubcoreMesh(
        core_axis_name="c", subcore_axis_name="s",
        num_cores=NUM_CORES, num_subcores=NUM_SUBCORES,
    )
    run = functools.partial(
        pl.kernel,
        out_type=jax.ShapeDtypeStruct((BATCH, OUT_W), jnp.float32),
        mesh=mesh,
        scratch_types=[
            pltpu.VMEM((BPW,), jnp.int32),
            pltpu.VMEM((1, PRE_W), jnp.float32),
            pltpu.VMEM((1, SUF_W), jnp.float32),
            pltpu.VMEM((CH, ROW), jnp.float32),
            pltpu.SemaphoreType.DMA,
        ],
    )(_body)
    out = run(lab, table, pfx, sfx)
    return out.reshape(BATCH, TOK_LEN, CTX_DIM)
